# baseline (speedup 1.0000x reference)
import functools

import jax
import jax.numpy as jnp
from jax import lax
from jax.experimental import pallas as pl
from jax.experimental.pallas import tpu as pltpu
from jax.experimental.pallas import tpu_sc as plsc

DIM = 64
LANES = 16
IDX_CHUNK = 4096


def _encoder_call(B, V):
    info = plsc.get_sparse_core_info()
    nw = info.num_cores * info.num_subcores
    f_per_w = DIM // nw
    n_chunks = B // IDX_CHUNK
    mesh = plsc.VectorSubcoreMesh(core_axis_name="c", subcore_axis_name="s")

    @functools.partial(
        pl.kernel,
        mesh=mesh,
        out_type=jax.ShapeDtypeStruct((DIM, B), jnp.float32),
        compiler_params=pltpu.CompilerParams(use_tc_tiling_on_sc=True,
                                             needs_layout_passes=False),
        scratch_types=[
            pltpu.VMEM((V,), jnp.float32),
            pltpu.VMEM((B,), jnp.float32),
            pltpu.VMEM((2, IDX_CHUNK), jnp.int32),
            pltpu.SemaphoreType.DMA,
            pltpu.SemaphoreType.DMA,
        ],
    )
    def run(cat_h, col_h, fab_h, ct_h, co_h, fb_h, out_h, row, acc, ixb,
            sem, sem_i):
        wid = lax.axis_index("s") * info.num_cores + lax.axis_index("c")
        for fi in range(f_per_w):
            f = wid + fi * nw
            for t, (tbl, idx_h) in enumerate(
                    [(ct_h, cat_h), (co_h, col_h), (fb_h, fab_h)]):
                rcp = pltpu.async_copy(tbl.at[f], row, sem)
                cps = [pltpu.async_copy(
                    idx_h.at[pl.ds(ci * IDX_CHUNK, IDX_CHUNK)],
                    ixb.at[ci % 2], sem_i) for ci in range(2)]
                rcp.wait()
                for ci in range(n_chunks):
                    cps[ci].wait()

                    def gloop(k, carry, _t=t, _ci=ci):
                        iv = ixb[_ci % 2, pl.ds(k * LANES, LANES)]
                        g = plsc.load_gather(row, [iv])
                        o = pl.ds(_ci * IDX_CHUNK + k * LANES, LANES)
                        if _t == 0:
                            acc[o] = g
                        else:
                            plsc.addupdate(acc.at[o], g)
                        return carry

                    lax.fori_loop(0, IDX_CHUNK // LANES, gloop, 0,
                                  unroll=16)
                    if ci + 2 < n_chunks:
                        cps.append(pltpu.async_copy(
                            idx_h.at[pl.ds((ci + 2) * IDX_CHUNK, IDX_CHUNK)],
                            ixb.at[ci % 2], sem_i))
            pltpu.sync_copy(acc, out_h.at[f])

    return run


def kernel(cat, col, fab, cat_table, col_table, fab_table):
    B = cat.shape[0]
    V = cat_table.shape[0]
    run = _encoder_call(B, V)
    out_t = run(cat.astype(jnp.int32), col.astype(jnp.int32),
                fab.astype(jnp.int32),
                cat_table.T, col_table.T, fab_table.T)
    return out_t.T


# speedup vs baseline: 1.5797x; 1.5797x over previous
import functools

import jax
import jax.numpy as jnp
from jax import lax
from jax.experimental import pallas as pl
from jax.experimental.pallas import tpu as pltpu
from jax.experimental.pallas import tpu_sc as plsc

DIM = 64
LANES = 16
IDX_CHUNK = 4096


def _encoder_call(B, V):
    info = plsc.get_sparse_core_info()
    nw = info.num_cores * info.num_subcores
    f_per_w = DIM // nw
    n_chunks = B // IDX_CHUNK
    mesh = plsc.VectorSubcoreMesh(core_axis_name="c", subcore_axis_name="s")

    @functools.partial(
        pl.kernel,
        mesh=mesh,
        out_type=jax.ShapeDtypeStruct((DIM, B), jnp.float32),
        compiler_params=pltpu.CompilerParams(use_tc_tiling_on_sc=True,
                                             needs_layout_passes=False),
        scratch_types=[
            pltpu.VMEM((V,), jnp.float32),
            pltpu.VMEM((B,), jnp.float32),
            pltpu.VMEM((2, IDX_CHUNK), jnp.int32),
            pltpu.SemaphoreType.DMA,
            pltpu.SemaphoreType.DMA,
        ],
    )
    def run(cat_h, col_h, fab_h, ct_h, co_h, fb_h, out_h, row, acc, ixb,
            sem, sem_i):
        wid = lax.axis_index("s") * info.num_cores + lax.axis_index("c")
        for fi in range(f_per_w):
            f = wid + fi * nw
            for t, (tbl, idx_h) in enumerate(
                    [(ct_h, cat_h), (co_h, col_h), (fb_h, fab_h)]):
                rcp = pltpu.async_copy(tbl.at[f], row, sem)
                cps = [pltpu.async_copy(
                    idx_h.at[pl.ds(ci * IDX_CHUNK, IDX_CHUNK)],
                    ixb.at[ci % 2], sem_i) for ci in range(2)]
                rcp.wait()
                for ci in range(n_chunks):
                    cps[ci].wait()

                    @plsc.parallel_loop(0, IDX_CHUNK, step=LANES,
                                        unroll=8)
                    def gloop(k, _t=t, _ci=ci):
                        iv = ixb[_ci % 2, pl.ds(k, LANES)]
                        g = plsc.load_gather(row, [iv])
                        o = pl.ds(_ci * IDX_CHUNK + k, LANES)
                        if _t == 0:
                            acc[o] = g
                        else:
                            plsc.addupdate(acc.at[o], g)
                    if ci + 2 < n_chunks:
                        cps.append(pltpu.async_copy(
                            idx_h.at[pl.ds((ci + 2) * IDX_CHUNK, IDX_CHUNK)],
                            ixb.at[ci % 2], sem_i))
            pltpu.sync_copy(acc, out_h.at[f])

    return run


def kernel(cat, col, fab, cat_table, col_table, fab_table):
    B = cat.shape[0]
    V = cat_table.shape[0]
    run = _encoder_call(B, V)
    out_t = run(cat.astype(jnp.int32), col.astype(jnp.int32),
                fab.astype(jnp.int32),
                cat_table.T, col_table.T, fab_table.T)
    return out_t.T
